# Initial kernel scaffold; baseline (speedup 1.0000x reference)
import jax
import jax.numpy as jnp
from jax.experimental import pallas as pl


def kernel(pos_xyz, cel_mat, adj, sft, spc):
    raise NotImplementedError("write your pallas kernel here")


# speedup vs baseline: 1.3831x; 1.3831x over previous
import functools

import jax
import jax.numpy as jnp
from jax import lax
from jax.experimental import pallas as pl
from jax.experimental.pallas import tpu as pltpu
from jax.experimental.pallas import tpu_sc as plsc

N_PNT = 100000
N_EDGE = 6400000
RC2 = 36.0

L = 16
NW = 32
C = 2048
NCHUNK = N_EDGE // C
_mesh = plsc.VectorSubcoreMesh(core_axis_name="c", subcore_axis_name="s")


def _wid():
    return lax.axis_index("s") * 2 + lax.axis_index("c")


def _pass1_body(x_hbm, y_hbm, z_hbm, adj0_hbm, adj1_hbm, sft_hbm,
                cel_hbm,
                sod_hbm, counts_hbm,
                idx0_v, idx1_v, sft_v,
                xi_v, yi_v, zi_v, xj_v, yj_v, zj_v, sod_v,
                cel_v,
                acc_v, sem):
    wid = _wid()
    lo = (wid * NCHUNK) // NW
    hi = ((wid + 1) * NCHUNK) // NW
    pltpu.sync_copy(cel_hbm, cel_v)
    cel16 = cel_v[...]
    c00 = cel16[0]
    c10 = cel16[1]
    c20 = cel16[2]
    c01 = cel16[3]
    c11 = cel16[4]
    c21 = cel16[5]
    c02 = cel16[6]
    c12 = cel16[7]
    c22 = cel16[8]

    def chunk(g, carry):
        pltpu.sync_copy(adj0_hbm.at[g], idx0_v)
        pltpu.sync_copy(adj1_hbm.at[g], idx1_v)
        pltpu.sync_copy(sft_hbm.at[g], sft_v)
        ds = [
            pltpu.async_copy(x_hbm.at[idx0_v], xi_v, sem),
            pltpu.async_copy(y_hbm.at[idx0_v], yi_v, sem),
            pltpu.async_copy(z_hbm.at[idx0_v], zi_v, sem),
            pltpu.async_copy(x_hbm.at[idx1_v], xj_v, sem),
            pltpu.async_copy(y_hbm.at[idx1_v], yj_v, sem),
            pltpu.async_copy(z_hbm.at[idx1_v], zj_v, sem),
        ]
        for d in ds:
            d.wait()

        def vec(i, acc):
            sl = pl.ds(i * L, L)
            xi = xi_v[sl]
            yi = yi_v[sl]
            zi = zi_v[sl]
            xj = xj_v[sl]
            yj = yj_v[sl]
            zj = zj_v[sl]
            s16 = sft_v[sl]
            i0 = jnp.where(s16 >= 18, 2, jnp.where(s16 >= 9, 1, 0))
            r9 = s16 - i0 * 9
            i1 = jnp.where(r9 >= 6, 2, jnp.where(r9 >= 3, 1, 0))
            i2 = r9 - i1 * 3
            fx = (i0 - 1).astype(jnp.float32)
            fy = (i1 - 1).astype(jnp.float32)
            fz = (i2 - 1).astype(jnp.float32)
            shx = (fx * c00 + fy * c10) + fz * c20
            shy = (fx * c01 + fy * c11) + fz * c21
            shz = (fx * c02 + fy * c12) + fz * c22
            vx = (xj - xi) + shx
            vy = (yj - yi) + shy
            vz = (zj - zi) + shz
            sod = (vx * vx + vy * vy) + vz * vz
            sod_v[sl] = sod
            return acc + jnp.where(sod < RC2, 1, 0)

        acc = lax.fori_loop(0, C // L, vec, carry)
        pltpu.sync_copy(sod_v, sod_hbm.at[g])
        return acc

    acc = lax.fori_loop(lo, hi, chunk, jnp.zeros((L,), jnp.int32))
    acc_v[...] = acc
    pltpu.sync_copy(acc_v, counts_hbm.at[wid])


def _pass2_body(adj0_hbm, adj1_hbm, sft_hbm, sod_hbm, counts_hbm,
                adj01_out, sft_out, sod_out, nv_out,
                adj0_v, adj1_v, sftc_v, sod_v, cnts_v, dst_v, dste_v,
                sodo_v, nv_v, bnc_v, sem):
    wid = _wid()
    lo = (wid * NCHUNK) // NW
    hi = ((wid + 1) * NCHUNK) // NW
    pltpu.sync_copy(counts_hbm, cnts_v)
    bnc_v[pl.ds(0, L)] = jnp.full((L,), 0, jnp.int32)

    def _incl_prefix(x):
        for k in (1, 2, 4, 8):
            bnc_v[pl.ds(L, L)] = x
            x = x + bnc_v[pl.ds(L - k, L)]
        return x

    total = jnp.int32(0)
    base = jnp.int32(0)
    for w in range(NW):
        s_w = _incl_prefix(cnts_v[w, :])[L - 1]
        total = total + s_w
        base = base + jnp.where(jnp.int32(w) < wid, s_w, 0)

    @pl.when(wid == 0)
    def _():
        nv_v[...] = jnp.full((L,), total, jnp.int32)
        pltpu.sync_copy(nv_v, nv_out)

    lanes = lax.iota(jnp.int32, L)
    start_edge = lo * C
    v0 = base
    i0 = total + (start_edge - base)

    def chunk(g, carry):
        v_off, i_off = carry
        pltpu.sync_copy(adj0_hbm.at[g], adj0_v)
        pltpu.sync_copy(adj1_hbm.at[g], adj1_v)
        pltpu.sync_copy(sft_hbm.at[g], sftc_v)
        pltpu.sync_copy(sod_hbm.at[g], sod_v)

        def vec(i, c2):
            vo, io = c2
            sl = pl.ds(i * L, L)
            sod = sod_v[sl]
            m = sod < RC2
            mi = jnp.where(m, 1, 0)
            incl = _incl_prefix(mi)
            pfx = incl - mi
            s = incl[L - 1]
            dst = jnp.where(m, vo + pfx, io + (lanes - pfx))
            dst_v[sl] = dst
            dste_v[sl] = dst + N_EDGE
            sodo_v[sl] = jnp.where(m, sod, 0.0)
            return (vo + s, io + (L - s))

        v_off, i_off = lax.fori_loop(0, C // L, vec, (v_off, i_off))
        d0 = pltpu.async_copy(adj0_v, adj01_out.at[dst_v], sem)
        d1 = pltpu.async_copy(adj1_v, adj01_out.at[dste_v], sem)
        d2 = pltpu.async_copy(sftc_v, sft_out.at[dst_v], sem)
        d3 = pltpu.async_copy(sodo_v, sod_out.at[dst_v], sem)
        d0.wait()
        d1.wait()
        d2.wait()
        d3.wait()
        return (v_off, i_off)

    lax.fori_loop(lo, hi, chunk, (v0, i0))


_pass1 = functools.partial(
    pl.kernel,
    out_type=(
        jax.ShapeDtypeStruct((NCHUNK, C), jnp.float32),
        jax.ShapeDtypeStruct((NW, L), jnp.int32),
    ),
    mesh=_mesh,
    scratch_types=[
        pltpu.VMEM((C,), jnp.int32),
        pltpu.VMEM((C,), jnp.int32),
        pltpu.VMEM((C,), jnp.int32),
        pltpu.VMEM((C,), jnp.float32),
        pltpu.VMEM((C,), jnp.float32),
        pltpu.VMEM((C,), jnp.float32),
        pltpu.VMEM((C,), jnp.float32),
        pltpu.VMEM((C,), jnp.float32),
        pltpu.VMEM((C,), jnp.float32),
        pltpu.VMEM((C,), jnp.float32),
        pltpu.VMEM((L,), jnp.float32),
        pltpu.VMEM((L,), jnp.int32),
        pltpu.SemaphoreType.DMA,
    ],
)(_pass1_body)

_pass2 = functools.partial(
    pl.kernel,
    out_type=(
        jax.ShapeDtypeStruct((2 * N_EDGE,), jnp.int32),
        jax.ShapeDtypeStruct((N_EDGE,), jnp.int32),
        jax.ShapeDtypeStruct((N_EDGE,), jnp.float32),
        jax.ShapeDtypeStruct((L,), jnp.int32),
    ),
    mesh=_mesh,
    scratch_types=[
        pltpu.VMEM((C,), jnp.int32),
        pltpu.VMEM((C,), jnp.int32),
        pltpu.VMEM((C,), jnp.int32),
        pltpu.VMEM((C,), jnp.float32),
        pltpu.VMEM((NW, L), jnp.int32),
        pltpu.VMEM((C,), jnp.int32),
        pltpu.VMEM((C,), jnp.int32),
        pltpu.VMEM((C,), jnp.float32),
        pltpu.VMEM((L,), jnp.int32),
        pltpu.VMEM((2 * L,), jnp.int32),
        pltpu.SemaphoreType.DMA,
    ],
)(_pass2_body)


def kernel(pos_xyz, cel_mat, adj, sft, spc):
    x = pos_xyz[:, 0]
    y = pos_xyz[:, 1]
    z = pos_xyz[:, 2]
    cel_flat = jnp.pad(cel_mat.T.reshape(9), (0, 7))
    adj0 = adj[0].reshape(NCHUNK, C)
    adj1 = adj[1].reshape(NCHUNK, C)
    sft2 = sft.reshape(NCHUNK, C)
    sod_s, counts = _pass1(x, y, z, adj0, adj1, sft2, cel_flat)
    adj01, sft_f, sod_f, nv = _pass2(adj0, adj1, sft2, sod_s, counts)
    return adj01.reshape(2, N_EDGE), sft_f, sod_f, nv[0]
